# baseline (speedup 1.0000x reference)
import jax
import jax.numpy as jnp
from jax.experimental import pallas as pl
from jax.experimental.pallas import tpu as pltpu


_BM = 512


def _z1_kernel(v_ref, w1_ref, z1_ref):
    z1_ref[...] = jnp.dot(
        v_ref[...].astype(jnp.bfloat16),
        w1_ref[...].astype(jnp.bfloat16),
        preferred_element_type=jnp.float32,
    ).astype(jnp.bfloat16)


def _layer1_kernel(adj_ref, z1_ref, w2_ref, z2_ref):
    h = jnp.dot(
        adj_ref[...].astype(jnp.bfloat16),
        z1_ref[...],
        preferred_element_type=jnp.float32,
    )
    h = jnp.maximum(h, 0.0).astype(jnp.bfloat16)
    z2_ref[...] = jnp.dot(
        h, w2_ref[...].astype(jnp.bfloat16), preferred_element_type=jnp.float32
    ).astype(jnp.bfloat16)


def _layer2_kernel(adj_ref, z2_ref, wout_ref, bout_ref, out_ref):
    i = pl.program_id(0)
    h = jnp.dot(
        adj_ref[...].astype(jnp.bfloat16),
        z2_ref[...],
        preferred_element_type=jnp.float32,
    )
    h = jnp.maximum(h, 0.0)
    x = jnp.sum(h, axis=1)
    contrib = jnp.sum(wout_ref[...] * x[None, :], axis=1)

    @pl.when(i == 0)
    def _():
        out_ref[...] = bout_ref[...]

    out_ref[...] += contrib[None, :]


def kernel(v, adj, W1, W2, W_out, b_out):
    N, F_IN = v.shape
    H1 = W1.shape[1]
    H2 = W2.shape[1]
    LABEL = W_out.shape[0]

    z1 = pl.pallas_call(
        _z1_kernel,
        grid=(N // _BM,),
        in_specs=[
            pl.BlockSpec((_BM, F_IN), lambda i: (i, 0)),
            pl.BlockSpec((F_IN, H1), lambda i: (0, 0)),
        ],
        out_specs=pl.BlockSpec((_BM, H1), lambda i: (i, 0)),
        out_shape=jax.ShapeDtypeStruct((N, H1), jnp.bfloat16),
        compiler_params=pltpu.CompilerParams(
            dimension_semantics=("parallel",)
        ),
    )(v, W1)

    z2 = pl.pallas_call(
        _layer1_kernel,
        grid=(N // _BM,),
        in_specs=[
            pl.BlockSpec((_BM, N), lambda i: (i, 0)),
            pl.BlockSpec((N, H1), lambda i: (0, 0)),
            pl.BlockSpec((H1, H2), lambda i: (0, 0)),
        ],
        out_specs=pl.BlockSpec((_BM, H2), lambda i: (i, 0)),
        out_shape=jax.ShapeDtypeStruct((N, H2), jnp.bfloat16),
        compiler_params=pltpu.CompilerParams(
            dimension_semantics=("parallel",)
        ),
    )(adj, z1, W2)

    out2d = pl.pallas_call(
        _layer2_kernel,
        grid=(N // _BM,),
        in_specs=[
            pl.BlockSpec((_BM, N), lambda i: (i, 0)),
            pl.BlockSpec((N, H2), lambda i: (0, 0)),
            pl.BlockSpec((LABEL, _BM), lambda i: (0, i)),
            pl.BlockSpec((1, LABEL), lambda i: (0, 0)),
        ],
        out_specs=pl.BlockSpec((1, LABEL), lambda i: (0, 0)),
        out_shape=jax.ShapeDtypeStruct((1, LABEL), jnp.float32),
    )(adj, z2, W_out, b_out.reshape(1, LABEL))

    return out2d.reshape(LABEL)


# speedup vs baseline: 1.1689x; 1.1689x over previous
import jax
import jax.numpy as jnp
from jax.experimental import pallas as pl
from jax.experimental.pallas import tpu as pltpu


_BM = 512


def _fused_kernel(
    adj_ref, v_ref, w1_ref, w2_ref, wout_ref, bout_ref, out_ref, z1_ref, z2_ref
):
    j = pl.program_id(0)
    i = pl.program_id(1)

    @pl.when((j == 0) & (i == 0))
    def _():
        z1_ref[...] = jnp.dot(
            v_ref[...].astype(jnp.bfloat16),
            w1_ref[...].astype(jnp.bfloat16),
            preferred_element_type=jnp.float32,
        ).astype(jnp.bfloat16)

    @pl.when(j == 0)
    def _():
        h = jnp.dot(
            adj_ref[...].astype(jnp.bfloat16),
            z1_ref[...],
            preferred_element_type=jnp.float32,
        )
        h = jnp.maximum(h, 0.0).astype(jnp.bfloat16)
        z2_ref[pl.ds(i * _BM, _BM), :] = jnp.dot(
            h, w2_ref[...].astype(jnp.bfloat16), preferred_element_type=jnp.float32
        ).astype(jnp.bfloat16)

    @pl.when(j == 1)
    def _():
        h = jnp.dot(
            adj_ref[...].astype(jnp.bfloat16),
            z2_ref[...],
            preferred_element_type=jnp.float32,
        )
        h = jnp.maximum(h, 0.0)
        x = jnp.sum(h, axis=1)
        contrib = jnp.sum(wout_ref[...] * x[None, :], axis=1)

        @pl.when(i == 0)
        def _():
            out_ref[...] = bout_ref[...]

        out_ref[...] += contrib[None, :]


def kernel(v, adj, W1, W2, W_out, b_out):
    try:
        _d = jax.devices()
        print("[probe] n_devices=", len(_d), "num_cores=", getattr(_d[0], "num_cores", None), "kind=", _d[0].device_kind)
    except Exception as _e:
        print("[probe] failed:", _e)
    N, F_IN = v.shape
    H1 = W1.shape[1]
    H2 = W2.shape[1]
    LABEL = W_out.shape[0]

    out2d = pl.pallas_call(
        _fused_kernel,
        grid=(2, N // _BM),
        in_specs=[
            pl.BlockSpec((_BM, N), lambda j, i: (i, 0)),
            pl.BlockSpec((N, F_IN), lambda j, i: (0, 0)),
            pl.BlockSpec((F_IN, H1), lambda j, i: (0, 0)),
            pl.BlockSpec((H1, H2), lambda j, i: (0, 0)),
            pl.BlockSpec((LABEL, _BM), lambda j, i: (0, i)),
            pl.BlockSpec((1, LABEL), lambda j, i: (0, 0)),
        ],
        out_specs=pl.BlockSpec((1, LABEL), lambda j, i: (0, 0)),
        out_shape=jax.ShapeDtypeStruct((1, LABEL), jnp.float32),
        scratch_shapes=[
            pltpu.VMEM((N, H1), jnp.bfloat16),
            pltpu.VMEM((N, H2), jnp.bfloat16),
        ],
    )(adj, v, W1, W2, W_out, b_out.reshape(1, LABEL))

    return out2d.reshape(LABEL)
